# Initial kernel scaffold; baseline (speedup 1.0000x reference)
import jax
import jax.numpy as jnp
from jax.experimental import pallas as pl


def kernel(hidden_states, gate_weight, e_score_correction_bias, w_gate, w_up, w_down):
    raise NotImplementedError("write your pallas kernel here")


# speedup vs baseline: 2.3717x; 2.3717x over previous
import jax
import jax.numpy as jnp
from jax import lax
from jax.experimental import pallas as pl
from jax.experimental.pallas import tpu as pltpu

_T, _D, _E, _K, _FF, _G, _TG = 2048, 1024, 16, 8, 512, 4, 2
_EPG = _E // _G


def _router_body(x_ref, gw_ref, bias_ref, dw_ref):
    x = x_ref[...]
    gw = gw_ref[...]
    logits = lax.dot_general(
        x, gw, (((1,), (1,)), ((), ())),
        preferred_element_type=jnp.float32)
    scores = 1.0 / (1.0 + jnp.exp(-logits))
    sfc = scores + bias_ref[...]

    gsums = []
    for g in range(_G):
        c = [sfc[:, 4 * g + i:4 * g + i + 1] for i in range(_EPG)]
        hi01, lo01 = jnp.maximum(c[0], c[1]), jnp.minimum(c[0], c[1])
        hi23, lo23 = jnp.maximum(c[2], c[3]), jnp.minimum(c[2], c[3])
        top1 = jnp.maximum(hi01, hi23)
        second = jnp.maximum(jnp.minimum(hi01, hi23),
                             jnp.where(hi01 >= hi23, lo01, lo23))
        gsums.append(top1 + second)

    sel = []
    for g in range(_G):
        beats = jnp.zeros_like(gsums[0], dtype=jnp.int32)
        for j in range(_G):
            if j == g:
                continue
            b = (gsums[j] >= gsums[g]) if j < g else (gsums[j] > gsums[g])
            beats = beats + b.astype(jnp.int32)
        sel.append((beats < _TG).astype(jnp.float32))

    mask = jnp.concatenate(
        [sel[g] for g in range(_G) for _ in range(_EPG)], axis=1)
    w = mask * scores
    denom = jnp.sum(w, axis=1, keepdims=True) + 1e-20
    dw_ref[...] = w / denom


def _expert_body(x_ref, wg_ref, wu_ref, wd_ref, dw_ref, out_ref):
    e = pl.program_id(0)
    x = x_ref[...]
    wg = wg_ref[0].astype(jnp.bfloat16)
    wu = wu_ref[0].astype(jnp.bfloat16)
    wd = wd_ref[0].astype(jnp.bfloat16)
    g = lax.dot_general(x, wg, (((1,), (1,)), ((), ())),
                        preferred_element_type=jnp.float32)
    u = lax.dot_general(x, wu, (((1,), (1,)), ((), ())),
                        preferred_element_type=jnp.float32)
    h = (g / (1.0 + jnp.exp(-g))) * u
    o = lax.dot_general(h.astype(jnp.bfloat16), wd,
                        (((1,), (1,)), ((), ())),
                        preferred_element_type=jnp.float32)
    o = o * dw_ref[0]

    @pl.when(e == 0)
    def _():
        out_ref[...] = o

    @pl.when(e != 0)
    def _():
        out_ref[...] = out_ref[...] + o


def kernel(hidden_states, gate_weight, e_score_correction_bias,
           w_gate, w_up, w_down):
    x32 = hidden_states.astype(jnp.float32)
    dense_w = pl.pallas_call(
        _router_body,
        out_shape=jax.ShapeDtypeStruct((_T, _E), jnp.float32),
    )(x32, gate_weight, e_score_correction_bias.reshape(1, _E))

    dw_t = dense_w.T.reshape(_E, _T, 1)
    xb = x32.astype(jnp.bfloat16)
    out = pl.pallas_call(
        _expert_body,
        grid=(_E,),
        in_specs=[
            pl.BlockSpec((_T, _D), lambda e: (0, 0)),
            pl.BlockSpec((1, _FF, _D), lambda e: (e, 0, 0)),
            pl.BlockSpec((1, _FF, _D), lambda e: (e, 0, 0)),
            pl.BlockSpec((1, _D, _FF), lambda e: (e, 0, 0)),
            pl.BlockSpec((1, _T, 1), lambda e: (e, 0, 0)),
        ],
        out_specs=pl.BlockSpec((_T, _D), lambda e: (0, 0)),
        out_shape=jax.ShapeDtypeStruct((_T, _D), jnp.float32),
        compiler_params=pltpu.CompilerParams(
            dimension_semantics=("arbitrary",)),
    )(xb, w_gate, w_up, w_down, dw_t)
    return out
